# baseline (speedup 1.0000x reference)
import functools

import jax
import jax.numpy as jnp
from jax import lax
from jax.experimental import pallas as pl
from jax.experimental.pallas import tpu as pltpu
from jax.experimental.pallas import tpu_sc as plsc

_VOCAB = 1000
_D = 1000
_DPAD = 1024
_B = 1024
_T = 50
_IDXB = 104

_NC = 2
_NS = 16
_NW = _NC * _NS
_BPW = _B // _NW

_CH = 4
_NCH = 13
_NSLOT = 3
_IDXW = _BPW * _IDXB


def _chunk_rows(c):
    return _CH if c < _NCH - 1 else _T - _CH * (_NCH - 1)


def _compact_rows(buf_a, buf_b, sl, slot, c):
    lanes = lax.iota(jnp.int32, 16)

    def row(k, carry):
        t = c * _CH + k
        r = slot * _CH + k
        for j in range(62):
            buf_b[sl, 0, t, pl.ds(j * 16, 16)] = (
                buf_a[r, j // 8, pl.ds((j * 16) % 128, 16)])
        x = plsc.load_gather(
            buf_a,
            [jnp.full((16,), r, jnp.int32),
             jnp.full((16,), 7, jnp.int32),
             lanes + 88])
        plsc.store_scatter(
            buf_b,
            [jnp.full((16,), sl, jnp.int32),
             jnp.zeros((16,), jnp.int32),
             jnp.full((16,), t, jnp.int32),
             lanes + 984],
            x)
        return carry

    lax.fori_loop(0, _chunk_rows(c), row, 0)


def _body(table_hbm, idx_hbm, out_hbm, idx_v, buf_a, buf_b,
          g0, g1, g2, s0, s1):
    wid = lax.axis_index("s") * _NC + lax.axis_index("c")
    pltpu.sync_copy(idx_hbm.at[pl.ds(wid * _IDXW, _IDXW)], idx_v)

    gsem = (g0, g1, g2)
    ssem = (s0, s1)

    def gather_dma(bl, c, slot):
        n = _chunk_rows(c)
        return pltpu.make_async_copy(
            table_hbm.at[idx_v.at[pl.ds(bl * _IDXB + c * 8, n)]],
            buf_a.at[pl.ds(slot * _CH, n)],
            gsem[slot],
        )

    def scatter_dma(bl, sl):
        return pltpu.make_async_copy(
            buf_b.at[sl],
            out_hbm.at[pl.ds(wid * _BPW + bl, 1)],
            ssem[sl],
        )

    def do_b(bl, sl):
        @pl.when(bl >= 2)
        def _():
            scatter_dma(bl - 2, sl).wait()

        gather_dma(bl, 0, 0).start()
        gather_dma(bl, 1, 1).start()
        for c in range(_NCH):
            slot = c % _NSLOT
            gather_dma(bl, c, slot).wait()
            if c + 2 < _NCH:
                gather_dma(bl, c + 2, (c + 2) % _NSLOT).start()
            _compact_rows(buf_a, buf_b, sl, slot, c)

        scatter_dma(bl, sl).start()

    def pair(bb, carry):
        do_b(2 * bb, 0)
        do_b(2 * bb + 1, 1)
        return carry

    lax.fori_loop(0, _BPW // 2, pair, 0)

    scatter_dma(_BPW - 2, 0).wait()
    scatter_dma(_BPW - 1, 1).wait()


@functools.partial(
    pl.kernel,
    mesh=plsc.VectorSubcoreMesh(core_axis_name="c", subcore_axis_name="s"),
    compiler_params=pltpu.CompilerParams(needs_layout_passes=False),
    out_type=jax.ShapeDtypeStruct((_B, _T, _D), jnp.float32),
    scratch_types=[
        pltpu.VMEM((_IDXW,), jnp.int32),
        pltpu.VMEM((_NSLOT * _CH, 8, 128), jnp.float32),
        pltpu.VMEM((2, 1, _T, _D), jnp.float32),
        pltpu.SemaphoreType.DMA,
        pltpu.SemaphoreType.DMA,
        pltpu.SemaphoreType.DMA,
        pltpu.SemaphoreType.DMA,
        pltpu.SemaphoreType.DMA,
    ],
)
def _gather_rows(table_hbm, idx_hbm, out_hbm, idx_v, buf_a, buf_b, *sems):
    _body(table_hbm, idx_hbm, out_hbm, idx_v, buf_a, buf_b, *sems)


def kernel(idx, table):
    table3 = jnp.pad(table, ((0, 0), (0, _DPAD - _D))).reshape(_VOCAB, 8, 128)
    idx_p = jnp.pad(idx, ((0, 0), (0, _NCH * _CH - _T)))
    idx_p = idx_p.reshape(_B, _NCH, _CH)
    idx_p = jnp.pad(idx_p, ((0, 0), (0, 0), (0, 8 - _CH))).reshape(-1)
    return _gather_rows(table3, idx_p)


# speedup vs baseline: 1.0208x; 1.0208x over previous
import functools

import jax
import jax.numpy as jnp
from jax import lax
from jax.experimental import pallas as pl
from jax.experimental.pallas import tpu as pltpu
from jax.experimental.pallas import tpu_sc as plsc

_VOCAB = 1000
_D = 1000
_DPAD = 1024
_B = 1024
_T = 50
_TPAD = 56

_NC = 2
_NS = 16
_NW = _NC * _NS
_BPW = _B // _NW

_CSTART = (0, 16, 32)
_CROWS = (16, 16, 18)
_IDXW = _BPW * _TPAD


def _compact_chunk(buf_a, buf_b, c):
    lanes = lax.iota(jnp.int32, 16)

    def row(k, carry):
        t = _CSTART[c] + k
        for j in range(62):
            buf_b[0, t, pl.ds(j * 16, 16)] = (
                buf_a[_CSTART[c] + k, j // 8, pl.ds((j * 16) % 128, 16)])
        x = plsc.load_gather(
            buf_a,
            [jnp.full((16,), _CSTART[c] + k, jnp.int32),
             jnp.full((16,), 7, jnp.int32),
             lanes + 88])
        plsc.store_scatter(
            buf_b,
            [jnp.zeros((16,), jnp.int32),
             jnp.full((16,), t, jnp.int32),
             lanes + 984],
            x)
        return carry

    lax.fori_loop(0, _CROWS[c], row, 0)


def _body(table_hbm, idx_hbm, out_hbm, idx_v, buf_a, buf_b, g0, g1, g2, s0):
    wid = lax.axis_index("s") * _NC + lax.axis_index("c")
    pltpu.sync_copy(idx_hbm.at[pl.ds(wid * _IDXW, _IDXW)], idx_v)

    gsem = (g0, g1, g2)

    def gather_dma(bl, c):
        return pltpu.make_async_copy(
            table_hbm.at[idx_v.at[pl.ds(bl * _TPAD + _CSTART[c], _CROWS[c])]],
            buf_a.at[pl.ds(_CSTART[c], _CROWS[c])],
            gsem[c],
        )

    def scatter_dma(bl):
        return pltpu.make_async_copy(
            buf_b,
            out_hbm.at[pl.ds(wid * _BPW + bl, 1)],
            s0,
        )

    gather_dma(0, 0).start()
    gather_dma(0, 1).start()

    def do_b(bl, carry):
        gather_dma(bl, 0).wait()
        gather_dma(bl, 2).start()

        @pl.when(bl >= 1)
        def _():
            scatter_dma(bl - 1).wait()

        _compact_chunk(buf_a, buf_b, 0)

        @pl.when(bl + 1 < _BPW)
        def _():
            gather_dma(bl + 1, 0).start()

        gather_dma(bl, 1).wait()
        _compact_chunk(buf_a, buf_b, 1)

        @pl.when(bl + 1 < _BPW)
        def _():
            gather_dma(bl + 1, 1).start()

        gather_dma(bl, 2).wait()
        _compact_chunk(buf_a, buf_b, 2)

        scatter_dma(bl).start()
        return carry

    lax.fori_loop(0, _BPW, do_b, 0)
    scatter_dma(_BPW - 1).wait()


@functools.partial(
    pl.kernel,
    mesh=plsc.VectorSubcoreMesh(core_axis_name="c", subcore_axis_name="s"),
    compiler_params=pltpu.CompilerParams(needs_layout_passes=False),
    out_type=jax.ShapeDtypeStruct((_B, _T, _D), jnp.float32),
    scratch_types=[
        pltpu.VMEM((_IDXW,), jnp.int32),
        pltpu.VMEM((_T, 8, 128), jnp.float32),
        pltpu.VMEM((1, _T, _D), jnp.float32),
        pltpu.SemaphoreType.DMA,
        pltpu.SemaphoreType.DMA,
        pltpu.SemaphoreType.DMA,
        pltpu.SemaphoreType.DMA,
    ],
)
def _gather_rows(table_hbm, idx_hbm, out_hbm, idx_v, buf_a, buf_b, *sems):
    _body(table_hbm, idx_hbm, out_hbm, idx_v, buf_a, buf_b, *sems)


def kernel(idx, table):
    table3 = jnp.pad(table, ((0, 0), (0, _DPAD - _D))).reshape(_VOCAB, 8, 128)
    idx_p = jnp.pad(idx, ((0, 0), (0, _TPAD - _T))).reshape(-1)
    return _gather_rows(table3, idx_p)
